# baseline (speedup 1.0000x reference)
import functools

import jax
import jax.numpy as jnp
from jax import lax
from jax.experimental import pallas as pl
from jax.experimental.pallas import tpu as pltpu
from jax.experimental.pallas import tpu_sc as plsc

N = 10000
NPAD = 10240
E = 320000
D = 128
NC = 2
NS = 16
NW = NC * NS
CHUNK = 64
NCHUNK = 160
NPHASE = 16
PC = NCHUNK // NPHASE
EP = NW * NCHUNK * CHUNK
NBUF = 5
STRIPE = NPAD // NS
RB = 10240

_sc_mesh = plsc.VectorSubcoreMesh(core_axis_name="c", subcore_axis_name="s")


@functools.partial(
    pl.kernel,
    mesh=_sc_mesh,
    out_type=jax.ShapeDtypeStruct((NC * NPAD,), jnp.float32),
    scratch_types=[
        pltpu.VMEM((NCHUNK, CHUNK), jnp.int32),
        pltpu.VMEM((CHUNK,), jnp.float32),
        pltpu.VMEM_SHARED((NPAD,), jnp.float32),
        pltpu.SemaphoreType.DMA((8,)),
    ],
)
def _deg(dst_hbm, zeros_hbm, ones_hbm, out_hbm, dst_v, ones_v, deg_sp, sem):
    c = lax.axis_index("c")
    s = lax.axis_index("s")
    wid = c * NS + s
    pltpu.sync_copy(zeros_hbm, deg_sp.at[pl.ds(s * STRIPE, STRIPE)])
    pltpu.sync_copy(ones_hbm, ones_v)
    pltpu.sync_copy(dst_hbm.at[wid], dst_v)
    plsc.subcore_barrier()

    def body(k, carry):
        base = k * 8
        for j in range(8):
            pltpu.async_copy(ones_v, deg_sp.at[dst_v.at[base + j]],
                             sem.at[j], add=True)
        for j in range(8):
            pltpu.make_async_copy(ones_v, deg_sp.at[dst_v.at[base + j]],
                                  sem.at[j]).wait()
        return carry

    lax.fori_loop(0, NCHUNK // 8, body, 0)
    plsc.subcore_barrier()
    pltpu.sync_copy(
        deg_sp.at[pl.ds(s * STRIPE, STRIPE)],
        out_hbm.at[pl.ds(c * NPAD + s * STRIPE, STRIPE)],
    )


@functools.partial(
    pl.kernel,
    mesh=_sc_mesh,
    out_type=jax.ShapeDtypeStruct((NC * NPAD, D), jnp.float32),
    scratch_types=[
        pltpu.VMEM((2, PC, CHUNK), jnp.int32),
        pltpu.VMEM((2, PC, CHUNK), jnp.int32),
        pltpu.VMEM((NBUF, CHUNK, D), jnp.float32),
        pltpu.VMEM_SHARED((NPAD, D), jnp.float32),
        pltpu.SemaphoreType.DMA((NBUF,)),
        pltpu.SemaphoreType.DMA((NBUF,)),
        pltpu.SemaphoreType.DMA((2,)),
    ],
)
def _agg(g_hbm, src_hbm, dst_hbm, zrows_hbm, out_hbm,
         src_v, dst_v, rows_v, acc_sp, gsem, ssem, isem):
    c = lax.axis_index("c")
    s = lax.axis_index("s")
    wid = c * NS + s
    pltpu.sync_copy(zrows_hbm, acc_sp.at[pl.ds(s * STRIPE, STRIPE)])
    plsc.subcore_barrier()

    def stage(q):
        b = q % 2
        pltpu.async_copy(src_hbm.at[wid * NPHASE + q], src_v.at[b], isem.at[b])
        pltpu.async_copy(dst_hbm.at[wid * NPHASE + q], dst_v.at[b], isem.at[b])

    def stage_wait(q):
        b = q % 2
        pltpu.make_async_copy(src_hbm.at[wid * NPHASE + q], src_v.at[b],
                              isem.at[b]).wait()
        pltpu.make_async_copy(dst_hbm.at[wid * NPHASE + q], dst_v.at[b],
                              isem.at[b]).wait()

    def gather(b, i, j):
        pltpu.async_copy(g_hbm.at[src_v.at[b, i]], rows_v.at[j], gsem.at[j])

    def gather_wait(b, i, j):
        pltpu.make_async_copy(g_hbm.at[src_v.at[b, i]], rows_v.at[j],
                              gsem.at[j]).wait()

    def scatter(b, i, j):
        pltpu.async_copy(rows_v.at[j], acc_sp.at[dst_v.at[b, i]], ssem.at[j],
                         add=True)

    def scatter_wait(b, i, j):
        pltpu.make_async_copy(rows_v.at[j], acc_sp.at[dst_v.at[b, i]],
                              ssem.at[j]).wait()

    stage(0)
    stage_wait(0)
    if NPHASE > 1:
        stage(1)
    for j in range(NBUF):
        gather(0, j, j)

    for p in range(NPHASE):
        cur = p % 2

        def body(k, carry, cur=cur):
            base = k * NBUF
            for j in range(NBUF):
                gather_wait(cur, base + j, j)
                scatter(cur, base + j, j)
            for j in range(NBUF):
                scatter_wait(cur, base + j, j)
                gather(cur, base + NBUF + j, j)
            return carry

        lax.fori_loop(0, PC // NBUF - 1, body, 0)
        if p + 1 < NPHASE:
            stage_wait(p + 1)
        base = PC - NBUF
        for j in range(NBUF):
            gather_wait(cur, base + j, j)
            scatter(cur, base + j, j)
        for j in range(NBUF):
            scatter_wait(cur, base + j, j)
            if p + 1 < NPHASE:
                gather(1 - cur, j, j)
        if p + 2 < NPHASE:
            stage(p + 2)
    plsc.subcore_barrier()
    pltpu.sync_copy(
        acc_sp.at[pl.ds(s * STRIPE, STRIPE)],
        out_hbm.at[pl.ds(c * NPAD + s * STRIPE, STRIPE)],
    )


def _mmraw_body(x_ref, w_ref, h_ref):
    h_ref[...] = jnp.dot(x_ref[...], w_ref[...],
                         preferred_element_type=jnp.float32)


_mmraw = pl.pallas_call(
    _mmraw_body,
    grid=(NPAD // RB,),
    in_specs=[
        pl.BlockSpec((RB, D), lambda i: (i, 0)),
        pl.BlockSpec((D, D), lambda i: (0, 0)),
    ],
    out_specs=pl.BlockSpec((RB, D), lambda i: (i, 0)),
    out_shape=jax.ShapeDtypeStruct((NPAD, D), jnp.float32),
)


def _scale_body(degp_ref, h_ref, g_ref, dinv_ref):
    dinv = lax.rsqrt(degp_ref[0] + degp_ref[1] + 1.0)
    dinv_ref[...] = dinv
    g_ref[...] = h_ref[...] * dinv


_scale = pl.pallas_call(
    _scale_body,
    grid=(NPAD // RB,),
    in_specs=[
        pl.BlockSpec((2, RB, 1), lambda i: (0, i, 0)),
        pl.BlockSpec((RB, D), lambda i: (i, 0)),
    ],
    out_specs=[
        pl.BlockSpec((RB, D), lambda i: (i, 0)),
        pl.BlockSpec((RB, 1), lambda i: (i, 0)),
    ],
    out_shape=[
        jax.ShapeDtypeStruct((NPAD, D), jnp.float32),
        jax.ShapeDtypeStruct((NPAD, 1), jnp.float32),
    ],
)


def _mid_body(dinv_ref, accp_ref, g_ref, w_ref, b_ref, out_ref):
    dinv = dinv_ref[...]
    t = (accp_ref[0] + accp_ref[1] + g_ref[...]) * dinv + b_ref[...]
    y = jnp.maximum(t, 0.0)
    out_ref[...] = jnp.dot(y, w_ref[...],
                           preferred_element_type=jnp.float32) * dinv


_mid = pl.pallas_call(
    _mid_body,
    grid=(NPAD // RB,),
    in_specs=[
        pl.BlockSpec((RB, 1), lambda i: (i, 0)),
        pl.BlockSpec((2, RB, D), lambda i: (0, i, 0)),
        pl.BlockSpec((RB, D), lambda i: (i, 0)),
        pl.BlockSpec((D, D), lambda i: (0, 0)),
        pl.BlockSpec((1, D), lambda i: (0, 0)),
    ],
    out_specs=pl.BlockSpec((RB, D), lambda i: (i, 0)),
    out_shape=jax.ShapeDtypeStruct((NPAD, D), jnp.float32),
)


def _fin_body(dinv_ref, accp_ref, g_ref, b_ref, out_ref):
    dinv = dinv_ref[...]
    out_ref[...] = (accp_ref[0] + accp_ref[1] + g_ref[...]) * dinv + b_ref[...]


_fin = pl.pallas_call(
    _fin_body,
    grid=(NPAD // RB,),
    in_specs=[
        pl.BlockSpec((RB, 1), lambda i: (i, 0)),
        pl.BlockSpec((2, RB, D), lambda i: (0, i, 0)),
        pl.BlockSpec((RB, D), lambda i: (i, 0)),
        pl.BlockSpec((1, D), lambda i: (0, 0)),
    ],
    out_specs=pl.BlockSpec((RB, D), lambda i: (i, 0)),
    out_shape=jax.ShapeDtypeStruct((NPAD, D), jnp.float32),
)


@jax.jit
def _run(x, ei, W1, b1, W2, b2):
    pad = N + (jnp.arange(EP - E, dtype=jnp.int32) % (NPAD - N))
    src = jnp.concatenate([ei[0], pad]).reshape(NW * NPHASE, PC, CHUNK)
    dst = jnp.concatenate([ei[1], pad]).reshape(NW * NPHASE, PC, CHUNK)
    dstd = dst.reshape(NW, NCHUNK, CHUNK)
    xp = jnp.pad(x, ((0, NPAD - N), (0, 0)))
    zeros_stripe = jnp.zeros((STRIPE,), jnp.float32)
    zrows = jnp.zeros((STRIPE, D), jnp.float32)
    ones_chunk = jnp.ones((CHUNK,), jnp.float32)

    degp = _deg(dstd, zeros_stripe, ones_chunk).reshape(NC, NPAD, 1)
    h1 = _mmraw(xp, W1)
    g1, dinv = _scale(degp, h1)
    acc1 = _agg(g1, src, dst, zrows).reshape(NC, NPAD, D)
    g2 = _mid(dinv, acc1, g1, W2, b1.reshape(1, D))
    acc2 = _agg(g2, src, dst, zrows).reshape(NC, NPAD, D)
    out = _fin(dinv, acc2, g2, b2.reshape(1, D))
    return out[:N]


def kernel(x, edge_index, W1, b1, W2, b2):
    x = x.astype(jnp.float32)
    ei = edge_index.astype(jnp.int32)
    return _run(x, ei, W1.astype(jnp.float32), b1.astype(jnp.float32),
                W2.astype(jnp.float32), b2.astype(jnp.float32))


# speedup vs baseline: 1.0182x; 1.0182x over previous
import functools

import jax
import jax.numpy as jnp
from jax import lax
from jax.experimental import pallas as pl
from jax.experimental.pallas import tpu as pltpu
from jax.experimental.pallas import tpu_sc as plsc

N = 10000
NPAD = 10240
E = 320000
D = 128
NC = 2
NS = 16
NW = NC * NS
CHUNK = 64
NCHUNK = 160
NPHASE = 16
PC = NCHUNK // NPHASE
EP = NW * NCHUNK * CHUNK
NBUF = 5
STRIPE = NPAD // NS
RB = 5120

_sc_mesh = plsc.VectorSubcoreMesh(core_axis_name="c", subcore_axis_name="s")


@functools.partial(
    pl.kernel,
    mesh=_sc_mesh,
    out_type=jax.ShapeDtypeStruct((NC * NPAD,), jnp.float32),
    scratch_types=[
        pltpu.VMEM((NCHUNK, CHUNK), jnp.int32),
        pltpu.VMEM((CHUNK,), jnp.float32),
        pltpu.VMEM_SHARED((NPAD,), jnp.float32),
        pltpu.SemaphoreType.DMA((8,)),
    ],
)
def _deg(dst_hbm, zeros_hbm, ones_hbm, out_hbm, dst_v, ones_v, deg_sp, sem):
    c = lax.axis_index("c")
    s = lax.axis_index("s")
    wid = c * NS + s
    pltpu.sync_copy(zeros_hbm, deg_sp.at[pl.ds(s * STRIPE, STRIPE)])
    pltpu.sync_copy(ones_hbm, ones_v)
    pltpu.sync_copy(dst_hbm.at[wid], dst_v)
    plsc.subcore_barrier()

    def body(k, carry):
        base = k * 8
        for j in range(8):
            pltpu.async_copy(ones_v, deg_sp.at[dst_v.at[base + j]],
                             sem.at[j], add=True)
        for j in range(8):
            pltpu.make_async_copy(ones_v, deg_sp.at[dst_v.at[base + j]],
                                  sem.at[j]).wait()
        return carry

    lax.fori_loop(0, NCHUNK // 8, body, 0)
    plsc.subcore_barrier()
    pltpu.sync_copy(
        deg_sp.at[pl.ds(s * STRIPE, STRIPE)],
        out_hbm.at[pl.ds(c * NPAD + s * STRIPE, STRIPE)],
    )


@functools.partial(
    pl.kernel,
    mesh=_sc_mesh,
    out_type=jax.ShapeDtypeStruct((NC * NPAD, D), jnp.float32),
    scratch_types=[
        pltpu.VMEM((2, PC, CHUNK), jnp.int32),
        pltpu.VMEM((2, PC, CHUNK), jnp.int32),
        pltpu.VMEM((NBUF, CHUNK, D), jnp.float32),
        pltpu.VMEM_SHARED((NPAD, D), jnp.float32),
        pltpu.SemaphoreType.DMA((NBUF,)),
        pltpu.SemaphoreType.DMA((NBUF,)),
        pltpu.SemaphoreType.DMA((2,)),
    ],
)
def _agg(g_hbm, src_hbm, dst_hbm, zrows_hbm, out_hbm,
         src_v, dst_v, rows_v, acc_sp, gsem, ssem, isem):
    c = lax.axis_index("c")
    s = lax.axis_index("s")
    wid = c * NS + s
    pltpu.sync_copy(zrows_hbm, acc_sp.at[pl.ds(s * STRIPE, STRIPE)])
    plsc.subcore_barrier()

    def stage(q):
        b = q % 2
        pltpu.async_copy(src_hbm.at[wid * NPHASE + q], src_v.at[b], isem.at[b])
        pltpu.async_copy(dst_hbm.at[wid * NPHASE + q], dst_v.at[b], isem.at[b])

    def stage_wait(q):
        b = q % 2
        pltpu.make_async_copy(src_hbm.at[wid * NPHASE + q], src_v.at[b],
                              isem.at[b]).wait()
        pltpu.make_async_copy(dst_hbm.at[wid * NPHASE + q], dst_v.at[b],
                              isem.at[b]).wait()

    def gather(b, i, j):
        pltpu.async_copy(g_hbm.at[src_v.at[b, i]], rows_v.at[j], gsem.at[j])

    def gather_wait(b, i, j):
        pltpu.make_async_copy(g_hbm.at[src_v.at[b, i]], rows_v.at[j],
                              gsem.at[j]).wait()

    def scatter(b, i, j):
        pltpu.async_copy(rows_v.at[j], acc_sp.at[dst_v.at[b, i]], ssem.at[j],
                         add=True)

    def scatter_wait(b, i, j):
        pltpu.make_async_copy(rows_v.at[j], acc_sp.at[dst_v.at[b, i]],
                              ssem.at[j]).wait()

    stage(0)
    stage_wait(0)
    if NPHASE > 1:
        stage(1)
    for j in range(NBUF):
        gather(0, j, j)

    for p in range(NPHASE):
        cur = p % 2

        def body(k, carry, cur=cur):
            base = k * NBUF
            for j in range(NBUF):
                gather_wait(cur, base + j, j)
                scatter(cur, base + j, j)
            for j in range(NBUF):
                scatter_wait(cur, base + j, j)
                gather(cur, base + NBUF + j, j)
            return carry

        lax.fori_loop(0, PC // NBUF - 1, body, 0)
        if p + 1 < NPHASE:
            stage_wait(p + 1)
        base = PC - NBUF
        for j in range(NBUF):
            gather_wait(cur, base + j, j)
            scatter(cur, base + j, j)
        for j in range(NBUF):
            scatter_wait(cur, base + j, j)
            if p + 1 < NPHASE:
                gather(1 - cur, j, j)
        if p + 2 < NPHASE:
            stage(p + 2)
    plsc.subcore_barrier()
    pltpu.sync_copy(
        acc_sp.at[pl.ds(s * STRIPE, STRIPE)],
        out_hbm.at[pl.ds(c * NPAD + s * STRIPE, STRIPE)],
    )


def _mmraw_body(degp_ref, x_ref, w_ref, g_ref, dinv_ref):
    dinv = lax.rsqrt(degp_ref[0] + degp_ref[1] + 1.0)
    dinv_ref[...] = dinv
    h = jnp.dot(x_ref[...], w_ref[...], preferred_element_type=jnp.float32)
    g_ref[...] = h * dinv


_mmraw = pl.pallas_call(
    _mmraw_body,
    grid=(NPAD // RB,),
    in_specs=[
        pl.BlockSpec((2, RB, 1), lambda i: (0, i, 0)),
        pl.BlockSpec((RB, D), lambda i: (i, 0)),
        pl.BlockSpec((D, D), lambda i: (0, 0)),
    ],
    out_specs=[
        pl.BlockSpec((RB, D), lambda i: (i, 0)),
        pl.BlockSpec((RB, 1), lambda i: (i, 0)),
    ],
    out_shape=[
        jax.ShapeDtypeStruct((NPAD, D), jnp.float32),
        jax.ShapeDtypeStruct((NPAD, 1), jnp.float32),
    ],
)


def _scale_body(degp_ref, h_ref, g_ref, dinv_ref):
    dinv = lax.rsqrt(degp_ref[0] + degp_ref[1] + 1.0)
    dinv_ref[...] = dinv
    g_ref[...] = h_ref[...] * dinv


_scale = pl.pallas_call(
    _scale_body,
    grid=(NPAD // RB,),
    in_specs=[
        pl.BlockSpec((2, RB, 1), lambda i: (0, i, 0)),
        pl.BlockSpec((RB, D), lambda i: (i, 0)),
    ],
    out_specs=[
        pl.BlockSpec((RB, D), lambda i: (i, 0)),
        pl.BlockSpec((RB, 1), lambda i: (i, 0)),
    ],
    out_shape=[
        jax.ShapeDtypeStruct((NPAD, D), jnp.float32),
        jax.ShapeDtypeStruct((NPAD, 1), jnp.float32),
    ],
)


def _mid_body(dinv_ref, accp_ref, g_ref, w_ref, b_ref, out_ref):
    dinv = dinv_ref[...]
    t = (accp_ref[0] + accp_ref[1] + g_ref[...]) * dinv + b_ref[...]
    y = jnp.maximum(t, 0.0)
    out_ref[...] = jnp.dot(y, w_ref[...],
                           preferred_element_type=jnp.float32) * dinv


_mid = pl.pallas_call(
    _mid_body,
    grid=(NPAD // RB,),
    in_specs=[
        pl.BlockSpec((RB, 1), lambda i: (i, 0)),
        pl.BlockSpec((2, RB, D), lambda i: (0, i, 0)),
        pl.BlockSpec((RB, D), lambda i: (i, 0)),
        pl.BlockSpec((D, D), lambda i: (0, 0)),
        pl.BlockSpec((1, D), lambda i: (0, 0)),
    ],
    out_specs=pl.BlockSpec((RB, D), lambda i: (i, 0)),
    out_shape=jax.ShapeDtypeStruct((NPAD, D), jnp.float32),
)


def _fin_body(dinv_ref, accp_ref, g_ref, b_ref, out_ref):
    dinv = dinv_ref[...]
    out_ref[...] = (accp_ref[0] + accp_ref[1] + g_ref[...]) * dinv + b_ref[...]


_fin = pl.pallas_call(
    _fin_body,
    grid=(NPAD // RB,),
    in_specs=[
        pl.BlockSpec((RB, 1), lambda i: (i, 0)),
        pl.BlockSpec((2, RB, D), lambda i: (0, i, 0)),
        pl.BlockSpec((RB, D), lambda i: (i, 0)),
        pl.BlockSpec((1, D), lambda i: (0, 0)),
    ],
    out_specs=pl.BlockSpec((RB, D), lambda i: (i, 0)),
    out_shape=jax.ShapeDtypeStruct((NPAD, D), jnp.float32),
)


@jax.jit
def _run(x, ei, W1, b1, W2, b2):
    pad = N + (jnp.arange(EP - E, dtype=jnp.int32) % (NPAD - N))
    src = jnp.concatenate([ei[0], pad]).reshape(NW * NPHASE, PC, CHUNK)
    dst = jnp.concatenate([ei[1], pad]).reshape(NW * NPHASE, PC, CHUNK)
    dstd = dst.reshape(NW, NCHUNK, CHUNK)
    xp = jnp.pad(x, ((0, NPAD - N), (0, 0)))
    zeros_stripe = jnp.zeros((STRIPE,), jnp.float32)
    zrows = jnp.zeros((STRIPE, D), jnp.float32)
    ones_chunk = jnp.ones((CHUNK,), jnp.float32)

    degp = _deg(dstd, zeros_stripe, ones_chunk).reshape(NC, NPAD, 1)
    g1, dinv = _mmraw(degp, xp, W1)
    acc1 = _agg(g1, src, dst, zrows).reshape(NC, NPAD, D)
    g2 = _mid(dinv, acc1, g1, W2, b1.reshape(1, D))
    acc2 = _agg(g2, src, dst, zrows).reshape(NC, NPAD, D)
    out = _fin(dinv, acc2, g2, b2.reshape(1, D))
    return out[:N]


def kernel(x, edge_index, W1, b1, W2, b2):
    x = x.astype(jnp.float32)
    ei = edge_index.astype(jnp.int32)
    return _run(x, ei, W1.astype(jnp.float32), b1.astype(jnp.float32),
                W2.astype(jnp.float32), b2.astype(jnp.float32))


# speedup vs baseline: 1.0316x; 1.0131x over previous
import functools

import jax
import jax.numpy as jnp
from jax import lax
from jax.experimental import pallas as pl
from jax.experimental.pallas import tpu as pltpu
from jax.experimental.pallas import tpu_sc as plsc

N = 10000
NPAD = 10240
E = 320000
D = 128
NC = 2
NS = 16
NW = NC * NS
CHUNK = 64
NCHUNK = 160
NPHASE = 16
PC = NCHUNK // NPHASE
EP = NW * NCHUNK * CHUNK
NBUF = 5
STRIPE = NPAD // NS
RB = 5000

_sc_mesh = plsc.VectorSubcoreMesh(core_axis_name="c", subcore_axis_name="s")


@functools.partial(
    pl.kernel,
    mesh=_sc_mesh,
    out_type=jax.ShapeDtypeStruct((NC * NPAD,), jnp.float32),
    scratch_types=[
        pltpu.VMEM((NCHUNK, CHUNK), jnp.int32),
        pltpu.VMEM((CHUNK,), jnp.float32),
        pltpu.VMEM_SHARED((NPAD,), jnp.float32),
        pltpu.SemaphoreType.DMA((8,)),
    ],
)
def _deg(dst_hbm, zeros_hbm, ones_hbm, out_hbm, dst_v, ones_v, deg_sp, sem):
    c = lax.axis_index("c")
    s = lax.axis_index("s")
    wid = c * NS + s
    pltpu.sync_copy(zeros_hbm, deg_sp.at[pl.ds(s * STRIPE, STRIPE)])
    pltpu.sync_copy(ones_hbm, ones_v)
    pltpu.sync_copy(dst_hbm.at[wid], dst_v)
    plsc.subcore_barrier()

    def body(k, carry):
        base = k * 8
        for j in range(8):
            pltpu.async_copy(ones_v, deg_sp.at[dst_v.at[base + j]],
                             sem.at[j], add=True)
        for j in range(8):
            pltpu.make_async_copy(ones_v, deg_sp.at[dst_v.at[base + j]],
                                  sem.at[j]).wait()
        return carry

    lax.fori_loop(0, NCHUNK // 8, body, 0)
    plsc.subcore_barrier()
    pltpu.sync_copy(
        deg_sp.at[pl.ds(s * STRIPE, STRIPE)],
        out_hbm.at[pl.ds(c * NPAD + s * STRIPE, STRIPE)],
    )


@functools.partial(
    pl.kernel,
    mesh=_sc_mesh,
    out_type=jax.ShapeDtypeStruct((NC * NPAD, D), jnp.float32),
    scratch_types=[
        pltpu.VMEM((2, PC, CHUNK), jnp.int32),
        pltpu.VMEM((2, PC, CHUNK), jnp.int32),
        pltpu.VMEM((NBUF, CHUNK, D), jnp.float32),
        pltpu.VMEM_SHARED((NPAD, D), jnp.float32),
        pltpu.SemaphoreType.DMA((NBUF,)),
        pltpu.SemaphoreType.DMA((NBUF,)),
        pltpu.SemaphoreType.DMA((2,)),
    ],
)
def _agg(g_hbm, src_hbm, dst_hbm, zrows_hbm, out_hbm,
         src_v, dst_v, rows_v, acc_sp, gsem, ssem, isem):
    c = lax.axis_index("c")
    s = lax.axis_index("s")
    wid = c * NS + s
    pltpu.sync_copy(zrows_hbm, acc_sp.at[pl.ds(s * STRIPE, STRIPE)])
    plsc.subcore_barrier()

    def stage(q):
        b = q % 2
        pltpu.async_copy(src_hbm.at[wid * NPHASE + q], src_v.at[b], isem.at[b])
        pltpu.async_copy(dst_hbm.at[wid * NPHASE + q], dst_v.at[b], isem.at[b])

    def stage_wait(q):
        b = q % 2
        pltpu.make_async_copy(src_hbm.at[wid * NPHASE + q], src_v.at[b],
                              isem.at[b]).wait()
        pltpu.make_async_copy(dst_hbm.at[wid * NPHASE + q], dst_v.at[b],
                              isem.at[b]).wait()

    def gather(b, i, j):
        pltpu.async_copy(g_hbm.at[src_v.at[b, i]], rows_v.at[j], gsem.at[j])

    def gather_wait(b, i, j):
        pltpu.make_async_copy(g_hbm.at[src_v.at[b, i]], rows_v.at[j],
                              gsem.at[j]).wait()

    def scatter(b, i, j):
        pltpu.async_copy(rows_v.at[j], acc_sp.at[dst_v.at[b, i]], ssem.at[j],
                         add=True)

    def scatter_wait(b, i, j):
        pltpu.make_async_copy(rows_v.at[j], acc_sp.at[dst_v.at[b, i]],
                              ssem.at[j]).wait()

    stage(0)
    stage_wait(0)
    if NPHASE > 1:
        stage(1)
    for j in range(NBUF):
        gather(0, j, j)

    for p in range(NPHASE):
        cur = p % 2

        def body(k, carry, cur=cur):
            base = k * NBUF
            for j in range(NBUF):
                gather_wait(cur, base + j, j)
                scatter(cur, base + j, j)
            for j in range(NBUF):
                scatter_wait(cur, base + j, j)
                gather(cur, base + NBUF + j, j)
            return carry

        lax.fori_loop(0, PC // NBUF - 1, body, 0)
        if p + 1 < NPHASE:
            stage_wait(p + 1)
        base = PC - NBUF
        for j in range(NBUF):
            gather_wait(cur, base + j, j)
            scatter(cur, base + j, j)
        for j in range(NBUF):
            scatter_wait(cur, base + j, j)
            if p + 1 < NPHASE:
                gather(1 - cur, j, j)
        if p + 2 < NPHASE:
            stage(p + 2)
    plsc.subcore_barrier()
    pltpu.sync_copy(
        acc_sp.at[pl.ds(s * STRIPE, STRIPE)],
        out_hbm.at[pl.ds(c * NPAD + s * STRIPE, STRIPE)],
    )


def _mmraw_body(degp_ref, x_ref, w_ref, g_ref, dinv_ref):
    dinv = lax.rsqrt(degp_ref[0] + degp_ref[1] + 1.0)
    dinv_ref[...] = dinv
    h = jnp.dot(x_ref[...], w_ref[...], preferred_element_type=jnp.float32)
    g_ref[...] = h * dinv


_mmraw = pl.pallas_call(
    _mmraw_body,
    grid=(N // RB,),
    in_specs=[
        pl.BlockSpec((2, RB, 1), lambda i: (0, i, 0)),
        pl.BlockSpec((RB, D), lambda i: (i, 0)),
        pl.BlockSpec((D, D), lambda i: (0, 0)),
    ],
    out_specs=[
        pl.BlockSpec((RB, D), lambda i: (i, 0)),
        pl.BlockSpec((RB, 1), lambda i: (i, 0)),
    ],
    out_shape=[
        jax.ShapeDtypeStruct((N, D), jnp.float32),
        jax.ShapeDtypeStruct((N, 1), jnp.float32),
    ],
)


def _mid_body(dinv_ref, accp_ref, g_ref, w_ref, b_ref, out_ref):
    dinv = dinv_ref[...]
    t = (accp_ref[0] + accp_ref[1] + g_ref[...]) * dinv + b_ref[...]
    y = jnp.maximum(t, 0.0)
    out_ref[...] = jnp.dot(y, w_ref[...],
                           preferred_element_type=jnp.float32) * dinv


_mid = pl.pallas_call(
    _mid_body,
    grid=(N // RB,),
    in_specs=[
        pl.BlockSpec((RB, 1), lambda i: (i, 0)),
        pl.BlockSpec((2, RB, D), lambda i: (0, i, 0)),
        pl.BlockSpec((RB, D), lambda i: (i, 0)),
        pl.BlockSpec((D, D), lambda i: (0, 0)),
        pl.BlockSpec((1, D), lambda i: (0, 0)),
    ],
    out_specs=pl.BlockSpec((RB, D), lambda i: (i, 0)),
    out_shape=jax.ShapeDtypeStruct((N, D), jnp.float32),
)


def _fin_body(dinv_ref, accp_ref, g_ref, b_ref, out_ref):
    dinv = dinv_ref[...]
    out_ref[...] = (accp_ref[0] + accp_ref[1] + g_ref[...]) * dinv + b_ref[...]


_fin = pl.pallas_call(
    _fin_body,
    grid=(N // RB,),
    in_specs=[
        pl.BlockSpec((RB, 1), lambda i: (i, 0)),
        pl.BlockSpec((2, RB, D), lambda i: (0, i, 0)),
        pl.BlockSpec((RB, D), lambda i: (i, 0)),
        pl.BlockSpec((1, D), lambda i: (0, 0)),
    ],
    out_specs=pl.BlockSpec((RB, D), lambda i: (i, 0)),
    out_shape=jax.ShapeDtypeStruct((N, D), jnp.float32),
)


@jax.jit
def _run(x, ei, W1, b1, W2, b2):
    npad = jnp.arange(EP - E, dtype=jnp.int32)
    src = jnp.concatenate([ei[0], npad % N]).reshape(NW * NPHASE, PC, CHUNK)
    dst = jnp.concatenate([ei[1], N + npad % (NPAD - N)]).reshape(
        NW * NPHASE, PC, CHUNK)
    dstd = dst.reshape(NW, NCHUNK, CHUNK)
    zeros_stripe = jnp.zeros((STRIPE,), jnp.float32)
    zrows = jnp.zeros((STRIPE, D), jnp.float32)
    ones_chunk = jnp.ones((CHUNK,), jnp.float32)

    degp = _deg(dstd, zeros_stripe, ones_chunk).reshape(NC, NPAD, 1)
    g1, dinv = _mmraw(degp, x, W1)
    acc1 = _agg(g1, src, dst, zrows).reshape(NC, NPAD, D)
    g2 = _mid(dinv, acc1, g1, W2, b1.reshape(1, D))
    acc2 = _agg(g2, src, dst, zrows).reshape(NC, NPAD, D)
    return _fin(dinv, acc2, g2, b2.reshape(1, D))


def kernel(x, edge_index, W1, b1, W2, b2):
    x = x.astype(jnp.float32)
    ei = edge_index.astype(jnp.int32)
    return _run(x, ei, W1.astype(jnp.float32), b1.astype(jnp.float32),
                W2.astype(jnp.float32), b2.astype(jnp.float32))


# speedup vs baseline: 1.0368x; 1.0051x over previous
import functools

import jax
import jax.numpy as jnp
from jax import lax
from jax.experimental import pallas as pl
from jax.experimental.pallas import tpu as pltpu
from jax.experimental.pallas import tpu_sc as plsc

N = 10000
NPAD = 10240
E = 320000
D = 128
NC = 2
NS = 16
NW = NC * NS
CHUNK = 80
NCHUNK = 128
NPHASE = 16
PC = NCHUNK // NPHASE
EP = NW * NCHUNK * CHUNK
NBUF = 4
STRIPE = NPAD // NS
RB = 5000

_sc_mesh = plsc.VectorSubcoreMesh(core_axis_name="c", subcore_axis_name="s")


@functools.partial(
    pl.kernel,
    mesh=_sc_mesh,
    out_type=jax.ShapeDtypeStruct((NC * NPAD,), jnp.float32),
    scratch_types=[
        pltpu.VMEM((NCHUNK, CHUNK), jnp.int32),
        pltpu.VMEM((CHUNK,), jnp.float32),
        pltpu.VMEM_SHARED((NPAD,), jnp.float32),
        pltpu.SemaphoreType.DMA((8,)),
    ],
)
def _deg(dst_hbm, zeros_hbm, ones_hbm, out_hbm, dst_v, ones_v, deg_sp, sem):
    c = lax.axis_index("c")
    s = lax.axis_index("s")
    wid = c * NS + s
    pltpu.sync_copy(zeros_hbm, deg_sp.at[pl.ds(s * STRIPE, STRIPE)])
    pltpu.sync_copy(ones_hbm, ones_v)
    pltpu.sync_copy(dst_hbm.at[wid], dst_v)
    plsc.subcore_barrier()

    def body(k, carry):
        base = k * 8
        for j in range(8):
            pltpu.async_copy(ones_v, deg_sp.at[dst_v.at[base + j]],
                             sem.at[j], add=True)
        for j in range(8):
            pltpu.make_async_copy(ones_v, deg_sp.at[dst_v.at[base + j]],
                                  sem.at[j]).wait()
        return carry

    lax.fori_loop(0, NCHUNK // 8, body, 0)
    plsc.subcore_barrier()
    pltpu.sync_copy(
        deg_sp.at[pl.ds(s * STRIPE, STRIPE)],
        out_hbm.at[pl.ds(c * NPAD + s * STRIPE, STRIPE)],
    )


@functools.partial(
    pl.kernel,
    mesh=_sc_mesh,
    out_type=jax.ShapeDtypeStruct((NC * NPAD, D), jnp.float32),
    scratch_types=[
        pltpu.VMEM((2, PC, CHUNK), jnp.int32),
        pltpu.VMEM((2, PC, CHUNK), jnp.int32),
        pltpu.VMEM((NBUF, CHUNK, D), jnp.float32),
        pltpu.VMEM_SHARED((NPAD, D), jnp.float32),
        pltpu.SemaphoreType.DMA((NBUF,)),
        pltpu.SemaphoreType.DMA((NBUF,)),
        pltpu.SemaphoreType.DMA((2,)),
    ],
)
def _agg(g_hbm, src_hbm, dst_hbm, zrows_hbm, out_hbm,
         src_v, dst_v, rows_v, acc_sp, gsem, ssem, isem):
    c = lax.axis_index("c")
    s = lax.axis_index("s")
    wid = c * NS + s
    pltpu.sync_copy(zrows_hbm, acc_sp.at[pl.ds(s * STRIPE, STRIPE)])
    plsc.subcore_barrier()

    def stage(q):
        b = q % 2
        pltpu.async_copy(src_hbm.at[wid * NPHASE + q], src_v.at[b], isem.at[b])
        pltpu.async_copy(dst_hbm.at[wid * NPHASE + q], dst_v.at[b], isem.at[b])

    def stage_wait(q):
        b = q % 2
        pltpu.make_async_copy(src_hbm.at[wid * NPHASE + q], src_v.at[b],
                              isem.at[b]).wait()
        pltpu.make_async_copy(dst_hbm.at[wid * NPHASE + q], dst_v.at[b],
                              isem.at[b]).wait()

    def gather(b, i, j):
        pltpu.async_copy(g_hbm.at[src_v.at[b, i]], rows_v.at[j], gsem.at[j])

    def gather_wait(b, i, j):
        pltpu.make_async_copy(g_hbm.at[src_v.at[b, i]], rows_v.at[j],
                              gsem.at[j]).wait()

    def scatter(b, i, j):
        pltpu.async_copy(rows_v.at[j], acc_sp.at[dst_v.at[b, i]], ssem.at[j],
                         add=True)

    def scatter_wait(b, i, j):
        pltpu.make_async_copy(rows_v.at[j], acc_sp.at[dst_v.at[b, i]],
                              ssem.at[j]).wait()

    stage(0)
    stage_wait(0)
    if NPHASE > 1:
        stage(1)
    for j in range(NBUF):
        gather(0, j, j)

    for p in range(NPHASE):
        cur = p % 2

        def body(k, carry, cur=cur):
            base = k * NBUF
            for j in range(NBUF):
                gather_wait(cur, base + j, j)
                scatter(cur, base + j, j)
            for j in range(NBUF):
                scatter_wait(cur, base + j, j)
                gather(cur, base + NBUF + j, j)
            return carry

        lax.fori_loop(0, PC // NBUF - 1, body, 0)
        if p + 1 < NPHASE:
            stage_wait(p + 1)
        base = PC - NBUF
        for j in range(NBUF):
            gather_wait(cur, base + j, j)
            scatter(cur, base + j, j)
        for j in range(NBUF):
            scatter_wait(cur, base + j, j)
            if p + 1 < NPHASE:
                gather(1 - cur, j, j)
        if p + 2 < NPHASE:
            stage(p + 2)
    plsc.subcore_barrier()
    pltpu.sync_copy(
        acc_sp.at[pl.ds(s * STRIPE, STRIPE)],
        out_hbm.at[pl.ds(c * NPAD + s * STRIPE, STRIPE)],
    )


def _mmraw_body(degp_ref, x_ref, w_ref, g_ref, dinv_ref):
    dinv = lax.rsqrt(degp_ref[0] + degp_ref[1] + 1.0)
    dinv_ref[...] = dinv
    h = jnp.dot(x_ref[...], w_ref[...], preferred_element_type=jnp.float32)
    g_ref[...] = h * dinv


_mmraw = pl.pallas_call(
    _mmraw_body,
    grid=(N // RB,),
    in_specs=[
        pl.BlockSpec((2, RB, 1), lambda i: (0, i, 0)),
        pl.BlockSpec((RB, D), lambda i: (i, 0)),
        pl.BlockSpec((D, D), lambda i: (0, 0)),
    ],
    out_specs=[
        pl.BlockSpec((RB, D), lambda i: (i, 0)),
        pl.BlockSpec((RB, 1), lambda i: (i, 0)),
    ],
    out_shape=[
        jax.ShapeDtypeStruct((N, D), jnp.float32),
        jax.ShapeDtypeStruct((N, 1), jnp.float32),
    ],
)


def _mid_body(dinv_ref, accp_ref, g_ref, w_ref, b_ref, out_ref):
    dinv = dinv_ref[...]
    t = (accp_ref[0] + accp_ref[1] + g_ref[...]) * dinv + b_ref[...]
    y = jnp.maximum(t, 0.0)
    out_ref[...] = jnp.dot(y, w_ref[...],
                           preferred_element_type=jnp.float32) * dinv


_mid = pl.pallas_call(
    _mid_body,
    grid=(N // RB,),
    in_specs=[
        pl.BlockSpec((RB, 1), lambda i: (i, 0)),
        pl.BlockSpec((2, RB, D), lambda i: (0, i, 0)),
        pl.BlockSpec((RB, D), lambda i: (i, 0)),
        pl.BlockSpec((D, D), lambda i: (0, 0)),
        pl.BlockSpec((1, D), lambda i: (0, 0)),
    ],
    out_specs=pl.BlockSpec((RB, D), lambda i: (i, 0)),
    out_shape=jax.ShapeDtypeStruct((N, D), jnp.float32),
)


def _fin_body(dinv_ref, accp_ref, g_ref, b_ref, out_ref):
    dinv = dinv_ref[...]
    out_ref[...] = (accp_ref[0] + accp_ref[1] + g_ref[...]) * dinv + b_ref[...]


_fin = pl.pallas_call(
    _fin_body,
    grid=(N // RB,),
    in_specs=[
        pl.BlockSpec((RB, 1), lambda i: (i, 0)),
        pl.BlockSpec((2, RB, D), lambda i: (0, i, 0)),
        pl.BlockSpec((RB, D), lambda i: (i, 0)),
        pl.BlockSpec((1, D), lambda i: (0, 0)),
    ],
    out_specs=pl.BlockSpec((RB, D), lambda i: (i, 0)),
    out_shape=jax.ShapeDtypeStruct((N, D), jnp.float32),
)


@jax.jit
def _run(x, ei, W1, b1, W2, b2):
    npad = jnp.arange(EP - E, dtype=jnp.int32)
    src = jnp.concatenate([ei[0], npad % N]).reshape(NW * NPHASE, PC, CHUNK)
    dst = jnp.concatenate([ei[1], N + npad % (NPAD - N)]).reshape(
        NW * NPHASE, PC, CHUNK)
    dstd = dst.reshape(NW, NCHUNK, CHUNK)
    zeros_stripe = jnp.zeros((STRIPE,), jnp.float32)
    zrows = jnp.zeros((STRIPE, D), jnp.float32)
    ones_chunk = jnp.ones((CHUNK,), jnp.float32)

    degp = _deg(dstd, zeros_stripe, ones_chunk).reshape(NC, NPAD, 1)
    g1, dinv = _mmraw(degp, x, W1)
    acc1 = _agg(g1, src, dst, zrows).reshape(NC, NPAD, D)
    g2 = _mid(dinv, acc1, g1, W2, b1.reshape(1, D))
    acc2 = _agg(g2, src, dst, zrows).reshape(NC, NPAD, D)
    return _fin(dinv, acc2, g2, b2.reshape(1, D))


def kernel(x, edge_index, W1, b1, W2, b2):
    x = x.astype(jnp.float32)
    ei = edge_index.astype(jnp.int32)
    return _run(x, ei, W1.astype(jnp.float32), b1.astype(jnp.float32),
                W2.astype(jnp.float32), b2.astype(jnp.float32))
